# baseline (speedup 1.0000x reference)
import jax
import jax.numpy as jnp
from jax.experimental import pallas as pl
from jax.experimental.pallas import tpu as pltpu

_E = 4
_K = 2
_EPS = 1e-3


def _silu_half(z):
    return z + z * jnp.tanh(z)


def _to_col(row):
    one = jnp.ones((1, 1), jnp.float32)
    return jax.lax.dot_general(row, one, (((0,), (0,)), ((), ())),
                               preferred_element_type=jnp.float32)


def _fused_kernel(x_ref, w1_ref, g1_ref, b1_ref, m1_ref, v1_ref,
                  wr_ref, br_ref, ce_ref, ge_ref, be_ref, me_ref, ve_ref,
                  w3_ref, g3_ref, b3_ref, m3_ref, v3_ref,
                  o_ref,
                  w1f_s, t1_s, wr_s, br_s, cef_s, tef_s, w3f_s, t3_s):
    Ch = w1_ref.shape[0]
    C1 = w1_ref.shape[1]

    @pl.when((pl.program_id(0) == 0) & (pl.program_id(1) == 0))
    def _fold():
        s1 = 0.5 * g1_ref[...] / jnp.sqrt(v1_ref[...] + _EPS)
        w1f_s[...] = w1_ref[...] * _to_col(s1)
        t1_s[...] = _to_col(0.5 * b1_ref[...] - m1_ref[...] * s1)
        wr_s[...] = jnp.zeros_like(wr_s)
        wr_s[0:_E, :] = wr_ref[...]
        br_s[...] = jnp.zeros_like(br_s)
        br_s[0:_E, :] = _to_col(br_ref[...])
        se = 0.5 * ge_ref[...] / jnp.sqrt(ve_ref[...] + _EPS)
        te = 0.5 * be_ref[...] - me_ref[...] * se
        for e in range(_E):
            sec = _to_col(se[e:e + 1, :])
            cef_s[e * Ch:(e + 1) * Ch, :] = (
                ce_ref[e * Ch:(e + 1) * Ch, :] * sec).astype(jnp.bfloat16)
            tef_s[e * Ch:(e + 1) * Ch, :] = _to_col(
                te[e:e + 1, :]).astype(jnp.bfloat16)
        s3 = 0.5 * g3_ref[...] / jnp.sqrt(v3_ref[...] + _EPS)
        w3f_s[...] = (w3_ref[...] * _to_col(s3)).astype(jnp.bfloat16)
        t3_s[...] = _to_col(0.5 * b3_ref[...] - m3_ref[...] * s3)

    xb3 = x_ref[...]
    _, Hb, W = xb3.shape
    xb = xb3.reshape(C1, Hb * W)

    h = _silu_half(jnp.dot(w1f_s[...], xb, preferred_element_type=jnp.float32)
                   + t1_s[...])

    logits = jnp.dot(wr_s[...], h, preferred_element_type=jnp.float32) + br_s[...]

    rows = [logits[e:e + 1, :] for e in range(_E)]
    m = jnp.maximum(jnp.maximum(rows[0], rows[1]), jnp.maximum(rows[2], rows[3]))
    ws = []
    for e in range(_E):
        rank = jnp.zeros_like(rows[e])
        for j in range(_E):
            if j == e:
                continue
            beats = (rows[j] > rows[e]) if j > e else (rows[j] >= rows[e])
            rank = rank + beats.astype(jnp.float32)
        sel = rank < float(_K)
        ws.append(jnp.where(sel, jnp.exp(rows[e] - m), 0.0))
    denom = ws[0] + ws[1] + ws[2] + ws[3]
    inv = 1.0 / denom

    h16 = h.astype(jnp.bfloat16)
    y = _silu_half(jnp.dot(cef_s[...], h16, preferred_element_type=jnp.float32
                           ).astype(jnp.bfloat16) + tef_s[...])
    acc = (ws[0] * inv).astype(jnp.bfloat16) * y[0 * Ch:1 * Ch]
    for e in range(1, _E):
        acc = acc + (ws[e] * inv).astype(jnp.bfloat16) * y[e * Ch:(e + 1) * Ch]

    res = _silu_half(jnp.dot(w3f_s[...], acc, preferred_element_type=jnp.float32)
                     + t3_s[...]) + xb
    o_ref[...] = res.reshape(C1, Hb, W)


def kernel(x, W1, bn1_g, bn1_b, bn1_m, bn1_v, Wr, br, We,
           bne_g, bne_b, bne_m, bne_v, W3, bn3_g, bn3_b, bn3_m, bn3_v):
    B, C1, H, W = x.shape
    Ch = W1.shape[0]
    Hb = 64
    x3 = x.reshape(B * C1, H, W)
    WeC = We[:, :, :, 1, 1].reshape(_E * Ch, Ch)

    grid = (B, H // Hb)
    full = lambda *s: pl.BlockSpec(s, lambda b, t: (0,) * len(s))
    f32 = jnp.float32
    bf16 = jnp.bfloat16
    out = pl.pallas_call(
        _fused_kernel,
        grid=grid,
        in_specs=[
            pl.BlockSpec((C1, Hb, W), lambda b, t: (b, t, 0)),
            full(Ch, C1),
            full(1, Ch), full(1, Ch), full(1, Ch), full(1, Ch),
            full(_E, Ch), full(1, _E),
            full(_E * Ch, Ch),
            full(_E, Ch), full(_E, Ch), full(_E, Ch), full(_E, Ch),
            full(C1, Ch),
            full(1, C1), full(1, C1), full(1, C1), full(1, C1),
        ],
        out_specs=pl.BlockSpec((C1, Hb, W), lambda b, t: (b, t, 0)),
        out_shape=jax.ShapeDtypeStruct((B * C1, H, W), f32),
        scratch_shapes=[
            pltpu.VMEM((Ch, C1), f32), pltpu.VMEM((Ch, 1), f32),
            pltpu.VMEM((8, Ch), f32), pltpu.VMEM((8, 1), f32),
            pltpu.VMEM((_E * Ch, Ch), bf16), pltpu.VMEM((_E * Ch, 1), bf16),
            pltpu.VMEM((C1, Ch), bf16), pltpu.VMEM((C1, 1), f32),
        ],
    )(x3, W1,
      bn1_g.reshape(1, Ch), bn1_b.reshape(1, Ch),
      bn1_m.reshape(1, Ch), bn1_v.reshape(1, Ch),
      Wr, br.reshape(1, _E), WeC,
      bne_g, bne_b, bne_m, bne_v,
      W3,
      bn3_g.reshape(1, C1), bn3_b.reshape(1, C1),
      bn3_m.reshape(1, C1), bn3_v.reshape(1, C1))
    return out.reshape(B, C1, H, W)


# speedup vs baseline: 1.0902x; 1.0902x over previous
import jax
import jax.numpy as jnp
from jax.experimental import pallas as pl
from jax.experimental.pallas import tpu as pltpu

_E = 4
_K = 2
_EPS = 1e-3


def _silu_half(z):
    return z + z * jnp.tanh(z)


def _to_col(row):
    one = jnp.ones((1, 1), jnp.float32)
    return jax.lax.dot_general(row, one, (((0,), (0,)), ((), ())),
                               preferred_element_type=jnp.float32)


def _fused_kernel(x_ref, w1_ref, g1_ref, b1_ref, m1_ref, v1_ref,
                  wr_ref, br_ref, ce_ref, ge_ref, be_ref, me_ref, ve_ref,
                  w3_ref, g3_ref, b3_ref, m3_ref, v3_ref,
                  o_ref,
                  w1f_s, t1_s, wr_s, br_s, cef_s, w3f_s):
    Ch = w1_ref.shape[0]
    C1 = w1_ref.shape[1]

    @pl.when((pl.program_id(0) == 0) & (pl.program_id(1) == 0))
    def _fold():
        s1 = 0.5 * g1_ref[...] / jnp.sqrt(v1_ref[...] + _EPS)
        w1f_s[...] = w1_ref[...] * _to_col(s1)
        t1_s[...] = _to_col(0.5 * b1_ref[...] - m1_ref[...] * s1)
        wr_s[...] = jnp.zeros_like(wr_s)
        wr_s[0:_E, :] = wr_ref[...]
        br_s[...] = jnp.zeros_like(br_s)
        br_s[0:_E, :] = _to_col(br_ref[...])
        se = 0.5 * ge_ref[...] / jnp.sqrt(ve_ref[...] + _EPS)
        te = 0.5 * be_ref[...] - me_ref[...] * se
        for e in range(_E):
            sec = _to_col(se[e:e + 1, :])
            cef_s[e * Ch:(e + 1) * Ch, 0:Ch] = (
                ce_ref[e * Ch:(e + 1) * Ch, :] * sec).astype(jnp.bfloat16)
            cef_s[e * Ch:(e + 1) * Ch, Ch:Ch + 1] = _to_col(
                te[e:e + 1, :]).astype(jnp.bfloat16)
        s3 = 0.5 * g3_ref[...] / jnp.sqrt(v3_ref[...] + _EPS)
        w3f_s[0:C1, 0:Ch] = (w3_ref[...] * _to_col(s3)).astype(jnp.bfloat16)
        w3f_s[0:C1, Ch:Ch + 1] = _to_col(
            0.5 * b3_ref[...] - m3_ref[...] * s3).astype(jnp.bfloat16)

    xb3 = x_ref[...]
    _, Hb, W = xb3.shape
    xb = xb3.reshape(C1, Hb * W)

    h = _silu_half(jnp.dot(w1f_s[...], xb, preferred_element_type=jnp.float32)
                   + t1_s[...])

    logits = jnp.dot(wr_s[...], h, preferred_element_type=jnp.float32) + br_s[...]

    rows = [logits[e:e + 1, :] for e in range(_E)]
    m = jnp.maximum(jnp.maximum(rows[0], rows[1]), jnp.maximum(rows[2], rows[3]))
    ws = []
    for e in range(_E):
        rank = jnp.zeros_like(rows[e])
        for j in range(_E):
            if j == e:
                continue
            beats = (rows[j] > rows[e]) if j > e else (rows[j] >= rows[e])
            rank = rank + beats.astype(jnp.float32)
        sel = rank < float(_K)
        ws.append(jnp.where(sel, jnp.exp(rows[e] - m), 0.0))
    denom = ws[0] + ws[1] + ws[2] + ws[3]
    inv = 1.0 / denom

    T = Hb * W
    ones_row = jnp.ones((1, T), jnp.bfloat16)
    hb = jnp.concatenate([h.astype(jnp.bfloat16), ones_row], axis=0)
    acc = None
    for e in range(_E):
        ye = _silu_half(
            jnp.dot(cef_s[e * Ch:(e + 1) * Ch, :], hb,
                    preferred_element_type=jnp.float32).astype(jnp.bfloat16))
        contrib = (ws[e] * inv).astype(jnp.bfloat16) * ye
        acc = contrib if acc is None else acc + contrib

    accb = jnp.concatenate([acc, ones_row], axis=0)
    res = _silu_half(jnp.dot(w3f_s[...], accb,
                             preferred_element_type=jnp.float32)) + xb
    o_ref[...] = res.reshape(C1, Hb, W)


def kernel(x, W1, bn1_g, bn1_b, bn1_m, bn1_v, Wr, br, We,
           bne_g, bne_b, bne_m, bne_v, W3, bn3_g, bn3_b, bn3_m, bn3_v):
    B, C1, H, W = x.shape
    Ch = W1.shape[0]
    Hb = 32
    x3 = x.reshape(B * C1, H, W)
    WeC = We[:, :, :, 1, 1].reshape(_E * Ch, Ch)

    grid = (B, H // Hb)
    full = lambda *s: pl.BlockSpec(s, lambda b, t: (0,) * len(s))
    f32 = jnp.float32
    bf16 = jnp.bfloat16
    out = pl.pallas_call(
        _fused_kernel,
        grid=grid,
        in_specs=[
            pl.BlockSpec((C1, Hb, W), lambda b, t: (b, t, 0)),
            full(Ch, C1),
            full(1, Ch), full(1, Ch), full(1, Ch), full(1, Ch),
            full(_E, Ch), full(1, _E),
            full(_E * Ch, Ch),
            full(_E, Ch), full(_E, Ch), full(_E, Ch), full(_E, Ch),
            full(C1, Ch),
            full(1, C1), full(1, C1), full(1, C1), full(1, C1),
        ],
        out_specs=pl.BlockSpec((C1, Hb, W), lambda b, t: (b, t, 0)),
        out_shape=jax.ShapeDtypeStruct((B * C1, H, W), f32),
        scratch_shapes=[
            pltpu.VMEM((Ch, C1), f32), pltpu.VMEM((Ch, 1), f32),
            pltpu.VMEM((8, Ch), f32), pltpu.VMEM((8, 1), f32),
            pltpu.VMEM((_E * Ch, Ch + 1), bf16),
            pltpu.VMEM((C1, Ch + 1), bf16),
        ],
    )(x3, W1,
      bn1_g.reshape(1, Ch), bn1_b.reshape(1, Ch),
      bn1_m.reshape(1, Ch), bn1_v.reshape(1, Ch),
      Wr, br.reshape(1, _E), WeC,
      bne_g, bne_b, bne_m, bne_v,
      W3,
      bn3_g.reshape(1, C1), bn3_b.reshape(1, C1),
      bn3_m.reshape(1, C1), bn3_v.reshape(1, C1))
    return out.reshape(B, C1, H, W)
